# Initial kernel scaffold; baseline (speedup 1.0000x reference)
import jax
import jax.numpy as jnp
from jax.experimental import pallas as pl


def kernel(x, expander_edge_index, expander_edge_attr, batch, WQ, bQ, WK, bK, WE, bE, WV, bV):
    raise NotImplementedError("write your pallas kernel here")


# speedup vs baseline: 18.2393x; 18.2393x over previous
import functools
import math

import jax
import jax.numpy as jnp
from jax import lax
from jax.experimental import pallas as pl
from jax.experimental.pallas import tpu as pltpu
from jax.experimental.pallas import tpu_sc as plsc

N_NODES = 10000
N_EDGES = 160000
IN_DIM = 256
OUT_DIM = 256
H = 8
DH = 32
DE = 16
HALF = 128
NC = 2
NS = 16
LANES = 16

EDGES_PER_TILE = N_EDGES // NS
EB = 80
NBLK = EDGES_PER_TILE // EB
NPAD = 10240
ROWS_PER_TILE = NPAD // NS
RCH = 32
NRCH = ROWS_PER_TILE // RCH

BN = 1000
BE_BLK = 2000


def _qkv_body(x_ref, w_ref, b_ref, q_ref, k_ref, v_ref):
    xb = x_ref[...]
    for m, o_ref in enumerate((q_ref, k_ref, v_ref)):
        acc = lax.dot_general(xb, w_ref[m, 0], (((1,), (1,)), ((), ())),
                              preferred_element_type=jnp.float32)
        o_ref[...] = acc + b_ref[m, 0, 0]


_qkv_call = pl.pallas_call(
    _qkv_body,
    grid=(N_NODES // BN, NC),
    in_specs=[
        pl.BlockSpec((BN, IN_DIM), lambda r, h: (r, 0)),
        pl.BlockSpec((3, 1, HALF, IN_DIM), lambda r, h: (0, h, 0, 0)),
        pl.BlockSpec((3, 1, 1, HALF), lambda r, h: (0, h, 0, 0)),
    ],
    out_specs=[
        pl.BlockSpec((BN, HALF), lambda r, h: (h * (N_NODES // BN) + r, 0))
        for _ in range(3)
    ],
    out_shape=[jax.ShapeDtypeStruct((NC * N_NODES, HALF), jnp.float32)
               for _ in range(3)],
)


def _eproj_body(a_ref, w_ref, b_ref, o_ref):
    acc = lax.dot_general(a_ref[...], w_ref[0], (((1,), (1,)), ((), ())),
                          preferred_element_type=jnp.float32)
    o_ref[...] = acc + b_ref[0, 0]


_eproj_call = pl.pallas_call(
    _eproj_body,
    grid=(N_EDGES // BE_BLK, NC),
    in_specs=[
        pl.BlockSpec((BE_BLK, DE), lambda r, h: (r, 0)),
        pl.BlockSpec((1, HALF, DE), lambda r, h: (h, 0, 0)),
        pl.BlockSpec((1, 1, HALF), lambda r, h: (h, 0, 0)),
    ],
    out_specs=pl.BlockSpec((BE_BLK, HALF),
                           lambda r, h: (h * (N_EDGES // BE_BLK) + r, 0)),
    out_shape=jax.ShapeDtypeStruct((NC * N_EDGES, HALF), jnp.float32),
)


_GATHER_DNUMS = lax.GatherDimensionNumbers(
    offset_dims=(), collapsed_slice_dims=(0,), start_index_map=(0,))


def _lane_permute(t, idx):
    return lax.gather(t, idx[:, None], _GATHER_DNUMS, slice_sizes=(1,),
                      mode=lax.GatherScatterMode.PROMISE_IN_BOUNDS)


def _sc_body(ktab, qtab, vtab, etab, src_hbm, dst_hbm, out_hbm,
             srcb, dstb, dadj, kb, qb, vb, eb, stage, acc):
    cid = lax.axis_index("c")
    sid = lax.axis_index("s")
    zeros16 = jnp.zeros((LANES,), jnp.float32)

    def _zrow(r, carry):
        for j in range(HALF // LANES):
            stage[r, pl.ds(j * LANES, LANES)] = zeros16
        return carry
    lax.fori_loop(0, RCH, _zrow, 0)
    for c in range(NRCH):
        pltpu.sync_copy(stage, acc.at[pl.ds(sid * ROWS_PER_TILE + c * RCH, RCH)])
    plsc.subcore_barrier()

    noff = cid * N_NODES
    eoff = cid * N_EDGES

    def _blk(b, carry):
        base = sid * EDGES_PER_TILE + b * EB
        pltpu.sync_copy(src_hbm.at[pl.ds(base, EB)], srcb)
        pltpu.sync_copy(dst_hbm.at[pl.ds(base, EB)], dstb)
        for j in range(EB // LANES):
            sl = pl.ds(j * LANES, LANES)
            srcb[sl] = srcb[sl] + noff
            dadj[sl] = dstb[sl] + noff
        pltpu.sync_copy(ktab.at[srcb], kb)
        pltpu.sync_copy(qtab.at[dadj], qb)
        pltpu.sync_copy(vtab.at[srcb], vb)
        pltpu.sync_copy(etab.at[pl.ds(eoff + base, EB)], eb)

        lane = lax.iota(jnp.int32, LANES)

        def _edge(i, icarry):
            for h in range(HALF // DH):
                s0 = pl.ds(h * DH, LANES)
                s1 = pl.ds(h * DH + LANES, LANES)
                t = (kb[i, s0] * qb[i, s0] * eb[i, s0]
                     + kb[i, s1] * qb[i, s1] * eb[i, s1])
                for step in (8, 4, 2, 1):
                    t = t + _lane_permute(t, lane ^ step)
                sv = jnp.exp(jnp.clip(t, -5.0, 5.0))
                vb[i, s0] = vb[i, s0] * sv
                vb[i, s1] = vb[i, s1] * sv
            return icarry
        lax.fori_loop(0, EB, _edge, 0)
        pltpu.sync_copy(vb, acc.at[dstb], add=True)
        return carry
    lax.fori_loop(0, NBLK, _blk, 0)

    plsc.subcore_barrier()
    for c in range(NRCH):
        r0 = sid * ROWS_PER_TILE + c * RCH
        pltpu.sync_copy(acc.at[pl.ds(r0, RCH)], stage)
        pltpu.sync_copy(stage, out_hbm.at[pl.ds(cid * NPAD + r0, RCH)])


_sc_mesh = plsc.VectorSubcoreMesh(core_axis_name="c", subcore_axis_name="s",
                                  num_cores=NC, num_subcores=NS)

_sc_call = pl.kernel(
    _sc_body,
    out_type=jax.ShapeDtypeStruct((NC * NPAD, HALF), jnp.float32),
    mesh=_sc_mesh,
    scratch_types=[
        pltpu.VMEM((EB,), jnp.int32),
        pltpu.VMEM((EB,), jnp.int32),
        pltpu.VMEM((EB,), jnp.int32),
        pltpu.VMEM((EB, HALF), jnp.float32),
        pltpu.VMEM((EB, HALF), jnp.float32),
        pltpu.VMEM((EB, HALF), jnp.float32),
        pltpu.VMEM((EB, HALF), jnp.float32),
        pltpu.VMEM((RCH, HALF), jnp.float32),
        pltpu.VMEM_SHARED((NPAD, HALF), jnp.float32),
    ],
)


def kernel(x, expander_edge_index, expander_edge_attr, batch,
           WQ, bQ, WK, bK, WE, bE, WV, bV):
    scale = 1.0 / math.sqrt(DH)
    w_stack = jnp.stack([WQ * scale, WK, WV]).reshape(3, NC, HALF, IN_DIM)
    b_stack = jnp.stack([bQ * scale, bK, bV]).reshape(3, NC, 1, HALF)
    qtab, ktab, vtab = _qkv_call(x, w_stack, b_stack)
    etab = _eproj_call(expander_edge_attr, WE.reshape(NC, HALF, DE),
                       bE.reshape(NC, 1, HALF))
    src = expander_edge_index[0]
    dst = expander_edge_index[1]
    out2 = _sc_call(ktab, qtab, vtab, etab, src, dst)
    return (out2.reshape(NC, NPAD, HALF)[:, :N_NODES]
            .transpose(1, 0, 2).reshape(N_NODES, OUT_DIM))


# speedup vs baseline: 22.9112x; 1.2561x over previous
import functools
import math

import jax
import jax.numpy as jnp
from jax import lax
from jax.experimental import pallas as pl
from jax.experimental.pallas import tpu as pltpu
from jax.experimental.pallas import tpu_sc as plsc

N_NODES = 10000
N_EDGES = 160000
IN_DIM = 256
OUT_DIM = 256
H = 8
DH = 32
DE = 16
HALF = 128
NC = 2
NS = 16
LANES = 16

EDGES_PER_TILE = N_EDGES // NS
EB = 80
NBLK = EDGES_PER_TILE // EB
NPAD = 10240
ROWS_PER_TILE = NPAD // NS
RCH = 32
NRCH = ROWS_PER_TILE // RCH

BN = 1000
BE_BLK = 2000


def _qkv_body(x_ref, w_ref, b_ref, q_ref, k_ref, v_ref):
    xb = x_ref[...]
    for m, o_ref in enumerate((q_ref, k_ref, v_ref)):
        acc = lax.dot_general(xb, w_ref[m, 0], (((1,), (1,)), ((), ())),
                              preferred_element_type=jnp.float32)
        o_ref[...] = acc + b_ref[m, 0, 0]


_qkv_call = pl.pallas_call(
    _qkv_body,
    grid=(N_NODES // BN, NC),
    in_specs=[
        pl.BlockSpec((BN, IN_DIM), lambda r, h: (r, 0)),
        pl.BlockSpec((3, 1, HALF, IN_DIM), lambda r, h: (0, h, 0, 0)),
        pl.BlockSpec((3, 1, 1, HALF), lambda r, h: (0, h, 0, 0)),
    ],
    out_specs=[
        pl.BlockSpec((BN, HALF), lambda r, h: (h * (N_NODES // BN) + r, 0))
        for _ in range(3)
    ],
    out_shape=[jax.ShapeDtypeStruct((NC * N_NODES, HALF), jnp.float32)
               for _ in range(3)],
)


def _eproj_body(a_ref, w_ref, b_ref, o_ref):
    acc = lax.dot_general(a_ref[...], w_ref[0], (((1,), (1,)), ((), ())),
                          preferred_element_type=jnp.float32)
    o_ref[...] = acc + b_ref[0, 0]


_eproj_call = pl.pallas_call(
    _eproj_body,
    grid=(N_EDGES // BE_BLK, NC),
    in_specs=[
        pl.BlockSpec((BE_BLK, DE), lambda r, h: (r, 0)),
        pl.BlockSpec((1, HALF, DE), lambda r, h: (h, 0, 0)),
        pl.BlockSpec((1, 1, HALF), lambda r, h: (h, 0, 0)),
    ],
    out_specs=pl.BlockSpec((BE_BLK, HALF),
                           lambda r, h: (h * (N_EDGES // BE_BLK) + r, 0)),
    out_shape=jax.ShapeDtypeStruct((NC * N_EDGES, HALF), jnp.float32),
)


_GATHER_DNUMS = lax.GatherDimensionNumbers(
    offset_dims=(), collapsed_slice_dims=(0,), start_index_map=(0,))


def _lane_permute(t, idx):
    return lax.gather(t, idx[:, None], _GATHER_DNUMS, slice_sizes=(1,),
                      mode=lax.GatherScatterMode.PROMISE_IN_BOUNDS)


def _sc_body(ktab, qtab, vtab, etab, src_hbm, dst_hbm, out_hbm,
             srcb, dstb, dadj, kb, qb, vb, eb, stage, acc,
             ksem, qsem, vsem, esem):
    cid = lax.axis_index("c")
    sid = lax.axis_index("s")
    zeros16 = jnp.zeros((LANES,), jnp.float32)

    def _zrow(r, carry):
        for j in range(HALF // LANES):
            stage[r, pl.ds(j * LANES, LANES)] = zeros16
        return carry
    lax.fori_loop(0, RCH, _zrow, 0)
    for c in range(NRCH):
        pltpu.sync_copy(stage, acc.at[pl.ds(sid * ROWS_PER_TILE + c * RCH, RCH)])
    plsc.subcore_barrier()

    noff = cid * N_NODES
    eoff = cid * N_EDGES

    def _blk(b, carry):
        base = sid * EDGES_PER_TILE + b * EB
        pltpu.sync_copy(src_hbm.at[pl.ds(base, EB)], srcb)
        pltpu.sync_copy(dst_hbm.at[pl.ds(base, EB)], dstb)
        for j in range(EB // LANES):
            sl = pl.ds(j * LANES, LANES)
            srcb[sl] = srcb[sl] + noff
            dadj[sl] = dstb[sl] + noff
        ck = pltpu.async_copy(ktab.at[srcb], kb, ksem)
        cq = pltpu.async_copy(qtab.at[dadj], qb, qsem)
        cv = pltpu.async_copy(vtab.at[srcb], vb, vsem)
        ce = pltpu.async_copy(etab.at[pl.ds(eoff + base, EB)], eb, esem)
        ck.wait()
        cq.wait()
        cv.wait()
        ce.wait()

        lane = lax.iota(jnp.int32, LANES)

        def _edge(i, icarry):
            for h in range(HALF // DH):
                s0 = pl.ds(h * DH, LANES)
                s1 = pl.ds(h * DH + LANES, LANES)
                t = (kb[i, s0] * qb[i, s0] * eb[i, s0]
                     + kb[i, s1] * qb[i, s1] * eb[i, s1])
                for step in (8, 4, 2, 1):
                    t = t + _lane_permute(t, lane ^ step)
                sv = jnp.exp(jnp.clip(t, -5.0, 5.0))
                vb[i, s0] = vb[i, s0] * sv
                vb[i, s1] = vb[i, s1] * sv
            return icarry
        lax.fori_loop(0, EB, _edge, 0)
        pltpu.sync_copy(vb, acc.at[dstb], add=True)
        return carry
    lax.fori_loop(0, NBLK, _blk, 0)

    plsc.subcore_barrier()
    for c in range(NRCH):
        r0 = sid * ROWS_PER_TILE + c * RCH
        pltpu.sync_copy(acc.at[pl.ds(r0, RCH)], stage)
        pltpu.sync_copy(stage, out_hbm.at[pl.ds(cid * NPAD + r0, RCH)])


_sc_mesh = plsc.VectorSubcoreMesh(core_axis_name="c", subcore_axis_name="s",
                                  num_cores=NC, num_subcores=NS)

_sc_call = pl.kernel(
    _sc_body,
    out_type=jax.ShapeDtypeStruct((NC * NPAD, HALF), jnp.float32),
    mesh=_sc_mesh,
    scratch_types=[
        pltpu.VMEM((EB,), jnp.int32),
        pltpu.VMEM((EB,), jnp.int32),
        pltpu.VMEM((EB,), jnp.int32),
        pltpu.VMEM((EB, HALF), jnp.float32),
        pltpu.VMEM((EB, HALF), jnp.float32),
        pltpu.VMEM((EB, HALF), jnp.float32),
        pltpu.VMEM((EB, HALF), jnp.float32),
        pltpu.VMEM((RCH, HALF), jnp.float32),
        pltpu.VMEM_SHARED((NPAD, HALF), jnp.float32),
        pltpu.SemaphoreType.DMA,
        pltpu.SemaphoreType.DMA,
        pltpu.SemaphoreType.DMA,
        pltpu.SemaphoreType.DMA,
    ],
)


def kernel(x, expander_edge_index, expander_edge_attr, batch,
           WQ, bQ, WK, bK, WE, bE, WV, bV):
    scale = 1.0 / math.sqrt(DH)
    w_stack = jnp.stack([WQ * scale, WK, WV]).reshape(3, NC, HALF, IN_DIM)
    b_stack = jnp.stack([bQ * scale, bK, bV]).reshape(3, NC, 1, HALF)
    qtab, ktab, vtab = _qkv_call(x, w_stack, b_stack)
    etab = _eproj_call(expander_edge_attr, WE.reshape(NC, HALF, DE),
                       bE.reshape(NC, 1, HALF))
    src = expander_edge_index[0]
    dst = expander_edge_index[1]
    out2 = _sc_call(ktab, qtab, vtab, etab, src, dst)
    return (out2.reshape(NC, NPAD, HALF)[:, :N_NODES]
            .transpose(1, 0, 2).reshape(N_NODES, OUT_DIM))
